# Initial kernel scaffold; baseline (speedup 1.0000x reference)
import functools

import jax
import jax.numpy as jnp
from jax import lax
from jax.experimental import pallas as pl
from jax.experimental.pallas import tpu as pltpu
from jax.experimental.pallas import tpu_sc as plsc

N, E = 50000, 800000
N1, E1 = 25000, 400000
N2, E2 = 12500, 200000
H = 64

B0, CH0 = 1024, 49
B1, CH1 = 1024, 25
B2, CH2 = 512, 25
BG1, CHG1 = 1024, 2
BG2, CHG2 = 512, 2
E0P = 16 * B0 * CH0
E1P = 16 * B1 * CH1
E2P = 16 * B2 * CH2
M1P = 16 * BG1 * CHG1
M2P = 16 * BG2 * CHG2
R0, R1, R2 = 50048, 25088, 12544

_MESH = plsc.VectorSubcoreMesh(core_axis_name="c", subcore_axis_name="s")


def _sc_segsum(table2, src, dst, zeros, *, rp, wh, b, ch, mid=None, addend=None):
    nz = rp // 16
    scratch = [
        pltpu.VMEM((b,), jnp.int32),
        pltpu.VMEM((b,), jnp.int32),
        pltpu.VMEM((b,), jnp.int32),
        pltpu.VMEM((b, wh), jnp.float32),
        pltpu.VMEM_SHARED((rp, wh), jnp.float32),
    ]
    has_add = addend is not None
    has_mid = mid is not None
    if has_add:
        scratch.append(pltpu.VMEM((b, wh), jnp.float32))
    if has_mid:
        scratch.append(pltpu.VMEM((mid.shape[0],), jnp.int32))

    def body(*refs):
        it = iter(refs)
        tbl = next(it); srcr = next(it); dstr = next(it); zr = next(it)
        addr = next(it) if has_add else None
        midr = next(it) if has_mid else None
        out = next(it)
        sbuf = next(it); dbuf = next(it); gbuf = next(it); rows = next(it)
        acc = next(it)
        abuf = next(it) if has_add else None
        midv = next(it) if has_mid else None
        c = lax.axis_index("c")
        s = lax.axis_index("s")
        pltpu.sync_copy(zr.at[pl.ds(s * nz, nz)], acc.at[pl.ds(s * nz, nz)])
        if has_mid:
            pltpu.sync_copy(midr, midv)
        plsc.subcore_barrier()

        def chunk(k, carry):
            base = (k * 16 + s) * b
            pltpu.sync_copy(srcr.at[pl.ds(base, b)], sbuf)
            pltpu.sync_copy(dstr.at[pl.ds(base, b)], dbuf)

            def ib(i, cc):
                v = sbuf[pl.ds(i * 16, 16)]
                if has_mid:
                    v = plsc.load_gather(midv, [v])
                gbuf[pl.ds(i * 16, 16)] = v * 2 + c
                return cc

            lax.fori_loop(0, b // 16, ib, 0, unroll=4)
            pltpu.sync_copy(tbl.at[gbuf], rows)
            pltpu.sync_copy(rows, acc.at[dbuf], add=True)
            if has_add:
                pltpu.sync_copy(addr.at[c, pl.ds(base, b)], abuf)
                pltpu.sync_copy(abuf, acc.at[dbuf], add=True)
            return carry

        lax.fori_loop(0, ch, chunk, 0)
        plsc.subcore_barrier()
        pltpu.sync_copy(acc.at[pl.ds(s * nz, nz)], out.at[c, pl.ds(s * nz, nz)])

    args = [table2, src, dst, zeros]
    if has_add:
        args.append(addend)
    if has_mid:
        args.append(mid)
    return pl.kernel(
        body,
        out_type=jax.ShapeDtypeStruct((2, rp, wh), jnp.float32),
        mesh=_MESH,
        scratch_types=scratch,
    )(*args)


def _sc_gather(table2, ids, *, wh, b, ch):
    mp = ids.shape[0]

    def body(tbl, idr, out, sbuf, gbuf, rows):
        c = lax.axis_index("c")
        s = lax.axis_index("s")

        def chunk(k, carry):
            base = (k * 16 + s) * b
            pltpu.sync_copy(idr.at[pl.ds(base, b)], sbuf)

            def ib(i, cc):
                gbuf[pl.ds(i * 16, 16)] = sbuf[pl.ds(i * 16, 16)] * 2 + c
                return cc

            lax.fori_loop(0, b // 16, ib, 0, unroll=4)
            pltpu.sync_copy(tbl.at[gbuf], rows)
            pltpu.sync_copy(rows, out.at[c, pl.ds(base, b)])
            return carry

        lax.fori_loop(0, ch, chunk, 0)

    return pl.kernel(
        body,
        out_type=jax.ShapeDtypeStruct((2, mp, wh), jnp.float32),
        mesh=_MESH,
        scratch_types=[
            pltpu.VMEM((b,), jnp.int32),
            pltpu.VMEM((b,), jnp.int32),
            pltpu.VMEM((b, wh), jnp.float32),
        ],
    )(table2, ids)


def _full(a):
    return pl.BlockSpec(a.shape, lambda i: (0,) * a.ndim)


def _ln(t):
    m = jnp.mean(t, axis=-1, keepdims=True)
    v = jnp.mean((t - m) ** 2, axis=-1, keepdims=True)
    return (t - m) * lax.rsqrt(v + 1e-5)


def _mm(a, b):
    return jnp.dot(a, b, preferred_element_type=jnp.float32)


def _tc_node(x, p):
    bn = 1000
    ws = [p['Wn1'], p['bn1'].reshape(1, H), p['Wn2'], p['bn2'].reshape(1, H),
          p['Wm1_0'], p['Ws1_0'], p['Wmk_0'], p['Wsk_0']]

    def body(xr, wn1, b1, wn2, b2, wm, wsx, wmk, wsk, hr, hmr, hsr, hmkr, hskr):
        h = _mm(xr[...], wn1[...]) + b1[...]
        h = jnp.where(h > 0, h, 0.01 * h)
        h = _ln(_mm(h, wn2[...]) + b2[...])
        hr[...] = h
        hmr[...] = _mm(h, wm[...])
        hsr[...] = _mm(h, wsx[...])
        hmkr[...] = _mm(h, wmk[...])
        hskr[...] = _mm(h, wsk[...])

    return pl.pallas_call(
        body,
        grid=(N // bn,),
        in_specs=[pl.BlockSpec((bn, 16), lambda i: (i, 0))] + [_full(w) for w in ws],
        out_specs=[pl.BlockSpec((bn, H), lambda i: (i, 0)),
                   pl.BlockSpec((bn, H), lambda i: (i, 0)),
                   pl.BlockSpec((bn, H), lambda i: (i, 0)),
                   pl.BlockSpec((bn, 2 * H), lambda i: (i, 0)),
                   pl.BlockSpec((bn, 2 * H), lambda i: (i, 0))],
        out_shape=[jax.ShapeDtypeStruct((N, H), jnp.float32),
                   jax.ShapeDtypeStruct((N, H), jnp.float32),
                   jax.ShapeDtypeStruct((N, H), jnp.float32),
                   jax.ShapeDtypeStruct((N, 2 * H), jnp.float32),
                   jax.ShapeDtypeStruct((N, 2 * H), jnp.float32)],
    )(x, *ws)


def _tc_edge(edge_attr, p):
    be = 2000
    ws = [p['We1'], p['be1'].reshape(1, H), p['We2'], p['be2'].reshape(1, H),
          p['Wed_0']]

    def body(ar, w1, b1, w2, b2, wed, outr):
        e = jnp.maximum(_mm(ar[...], w1[...]) + b1[...], 0.0)
        e = _ln(_mm(e, w2[...]) + b2[...])
        em = _mm(e, wed[...])
        outr[0] = em[:, :H // 2]
        outr[1] = em[:, H // 2:]

    return pl.pallas_call(
        body,
        grid=(E // be,),
        in_specs=[pl.BlockSpec((be, 4), lambda i: (i, 0))] + [_full(w) for w in ws],
        out_specs=pl.BlockSpec((2, be, H // 2), lambda i: (0, i, 0)),
        out_shape=jax.ShapeDtypeStruct((2, E, H // 2), jnp.float32),
    )(edge_attr, *ws)


def _tc_combine(agg, hs, wlist, odims, *, n, bn, wh):
    def body(*refs):
        ar, hsr = refs[0], refs[1]
        wrs = refs[2:2 + len(wlist)]
        hr = refs[2 + len(wlist)]
        outs = refs[3 + len(wlist):]
        h = jnp.maximum(jnp.concatenate([ar[0], ar[1]], axis=-1) + hsr[...], 0.0)
        hr[...] = h
        for wr, outr in zip(wrs, outs):
            outr[...] = _mm(h, wr[...])

    w2 = 2 * wh
    return pl.pallas_call(
        body,
        grid=(n // bn,),
        in_specs=[pl.BlockSpec((2, bn, wh), lambda i: (0, i, 0)),
                  pl.BlockSpec((bn, w2), lambda i: (i, 0))] +
                 [_full(w) for w in wlist],
        out_specs=[pl.BlockSpec((bn, w2), lambda i: (i, 0))] +
                  [pl.BlockSpec((bn, od), lambda i: (i, 0)) for od in odims],
        out_shape=[jax.ShapeDtypeStruct((n, w2), jnp.float32)] +
                  [jax.ShapeDtypeStruct((n, od), jnp.float32) for od in odims],
    )(agg, hs, *wlist)


def _tc_pool(aggk, gk, agg2, g2, *, n, bn, wh):
    w2 = 2 * wh

    def body(akr, gkr, a2r, g2r, tr, str_):
        i = pl.program_id(0)
        xs = jnp.maximum(jnp.concatenate([akr[0], akr[1]], -1) +
                         jnp.concatenate([gkr[0], gkr[1]], -1), 0.0)
        hb = jnp.maximum(jnp.concatenate([a2r[0], a2r[1]], -1) +
                         jnp.concatenate([g2r[0], g2r[1]], -1), 0.0)
        t = xs + hb
        tr[...] = t

        @pl.when(i == 0)
        def _():
            str_[...] = jnp.zeros((8, w2), jnp.float32)

        str_[0:1, :] += jnp.sum(t, axis=0, keepdims=True)
        str_[1:2, :] += jnp.sum(t * t, axis=0, keepdims=True)

    eb = pl.BlockSpec((2, bn, wh), lambda i: (0, i, 0))
    return pl.pallas_call(
        body,
        grid=(n // bn,),
        in_specs=[eb, eb, eb, eb],
        out_specs=[pl.BlockSpec((bn, w2), lambda i: (i, 0)),
                   pl.BlockSpec((8, w2), lambda i: (0, 0))],
        out_shape=[jax.ShapeDtypeStruct((n, w2), jnp.float32),
                   jax.ShapeDtypeStruct((8, w2), jnp.float32)],
    )(aggk, gk, agg2, g2)


def _tc_bn(t, st, wlist, odims, *, n, bn, w2):
    def body(*refs):
        tr, sr = refs[0], refs[1]
        wrs = refs[2:2 + len(wlist)]
        hr = refs[2 + len(wlist)]
        outs = refs[3 + len(wlist):]
        mean = sr[0:1, :] / n
        var = sr[1:2, :] / n - mean * mean
        h = jnp.maximum((tr[...] - mean) * lax.rsqrt(var + 1e-5), 0.0)
        hr[...] = h
        for wr, outr in zip(wrs, outs):
            outr[...] = _mm(h, wr[...])

    return pl.pallas_call(
        body,
        grid=(n // bn,),
        in_specs=[pl.BlockSpec((bn, w2), lambda i: (i, 0)),
                  pl.BlockSpec((8, w2), lambda i: (0, 0))] +
                 [_full(w) for w in wlist],
        out_specs=[pl.BlockSpec((bn, w2), lambda i: (i, 0))] +
                  [pl.BlockSpec((bn, od), lambda i: (i, 0)) for od in odims],
        out_shape=[jax.ShapeDtypeStruct((n, w2), jnp.float32)] +
                  [jax.ShapeDtypeStruct((n, od), jnp.float32) for od in odims],
    )(t, st, *wlist)


def _tc_head(aggb, hsb, eps, p):
    bn = 500
    ws = [p['Wl'], p['bl'].reshape(1, H), p['Wmu'], p['bmu'].reshape(1, H),
          p['Wlv'], p['blv'].reshape(1, H)]

    def body(ar, hsr, er, wl, bl, wmu, bmu, wlv, blv, klr, zr):
        h5 = jnp.maximum(jnp.concatenate([ar[0], ar[1]], -1) + hsr[...], 0.0)
        xt = _mm(h5, wl[...]) + bl[...]
        mu = _ln(jnp.maximum(_mm(xt, wmu[...]) + bmu[...], 0.0))
        lv = _ln(jnp.maximum(_mm(xt, wlv[...]) + blv[...], 0.0))
        zr[...] = mu + er[...] * jnp.exp(0.5 * lv)
        kl = jnp.mean(-0.5 * (1.0 + lv - mu * mu - jnp.exp(lv)), axis=1)
        klr[...] = kl.reshape(bn, 1)

    return pl.pallas_call(
        body,
        grid=(N2 // bn,),
        in_specs=[pl.BlockSpec((2, bn, 2 * H), lambda i: (0, i, 0)),
                  pl.BlockSpec((bn, 4 * H), lambda i: (i, 0)),
                  pl.BlockSpec((bn, H), lambda i: (i, 0))] +
                 [_full(w) for w in ws],
        out_specs=[pl.BlockSpec((bn, 1), lambda i: (i, 0)),
                   pl.BlockSpec((bn, H), lambda i: (i, 0))],
        out_shape=[jax.ShapeDtypeStruct((N2, 1), jnp.float32),
                   jax.ShapeDtypeStruct((N2, H), jnp.float32)],
    )(aggb, hsb, eps, *ws)


def _pad1(a, length, val):
    return jnp.concatenate(
        [a, jnp.full((length - a.shape[0],), val, a.dtype)])


def kernel(x, edge_attr, weights, params, edge_index, m_id0, m_g1, m_id1, m_g2):
    p = params
    src0 = _pad1(edge_index[0], E0P, 0)
    dst0 = _pad1(edge_index[1], E0P, N)
    src1 = _pad1(m_g1[0], E1P, 0)
    dst1 = _pad1(m_g1[1], E1P, N1)
    src2 = _pad1(m_g2[0], E2P, 0)
    dst2 = _pad1(m_g2[1], E2P, N2)
    mid0p = _pad1(m_id0, M1P, 0)
    mid1p = _pad1(m_id1, M2P, 0)
    z0 = jnp.zeros((R0, H // 2), jnp.float32)
    z1 = jnp.zeros((R1, H), jnp.float32)
    z2 = jnp.zeros((R2, 2 * H), jnp.float32)

    h, hm0, hs0, hmk, hsk = _tc_node(x, p)
    emsg = _tc_edge(edge_attr, p)

    agg0 = _sc_segsum(hm0.reshape(2 * N, H // 2), src0, dst0, z0,
                      rp=R0, wh=H // 2, b=B0, ch=CH0, addend=emsg)
    h1, hm2, hs2 = _tc_combine(
        agg0, hs0, [p['Wm2_0'], p['Ws2_0']], [2 * H, 2 * H],
        n=N, bn=1000, wh=H // 2)

    aggk = _sc_segsum(hmk.reshape(2 * N, H), src1, dst1, z1,
                      rp=R1, wh=H, b=B1, ch=CH1, mid=m_id0)
    agg2 = _sc_segsum(hm2.reshape(2 * N, H), src1, dst1, z1,
                      rp=R1, wh=H, b=B1, ch=CH1, mid=m_id0)
    gk = _sc_gather(hsk.reshape(2 * N, H), mid0p, wh=H, b=BG1, ch=CHG1)
    g2 = _sc_gather(hs2.reshape(2 * N, H), mid0p, wh=H, b=BG1, ch=CHG1)
    t, st = _tc_pool(aggk, gk, agg2, g2, n=N1, bn=1000, wh=H)
    h2, hm11, hs11, hmk1, hsk1 = _tc_bn(
        t, st, [p['Wm1_1'], p['Ws1_1'], p['Wmk_1'], p['Wsk_1']],
        [2 * H, 2 * H, 4 * H, 4 * H], n=N1, bn=1000, w2=2 * H)

    agg11 = _sc_segsum(hm11.reshape(2 * N1, H), src1, dst1, z1,
                       rp=R1, wh=H, b=B1, ch=CH1)
    h3, hm21, hs21 = _tc_combine(
        agg11, hs11, [p['Wm2_1'], p['Ws2_1']], [4 * H, 4 * H],
        n=N1, bn=1000, wh=H)

    aggk1 = _sc_segsum(hmk1.reshape(2 * N1, 2 * H), src2, dst2, z2,
                       rp=R2, wh=2 * H, b=B2, ch=CH2, mid=m_id1)
    agg21 = _sc_segsum(hm21.reshape(2 * N1, 2 * H), src2, dst2, z2,
                       rp=R2, wh=2 * H, b=B2, ch=CH2, mid=m_id1)
    gk1 = _sc_gather(hsk1.reshape(2 * N1, 2 * H), mid1p, wh=2 * H, b=BG2, ch=CHG2)
    g21 = _sc_gather(hs21.reshape(2 * N1, 2 * H), mid1p, wh=2 * H, b=BG2, ch=CHG2)
    t2, st2 = _tc_pool(aggk1, gk1, agg21, g21, n=N2, bn=500, wh=2 * H)
    h4, hmb, hsb = _tc_bn(
        t2, st2, [p['Wmb'], p['Wsb']], [4 * H, 4 * H], n=N2, bn=500, w2=4 * H)

    aggb = _sc_segsum(hmb.reshape(2 * N2, 2 * H), src2, dst2, z2,
                      rp=R2, wh=2 * H, b=B2, ch=CH2)

    eps = jax.random.normal(jax.random.key(42), (N2, H), jnp.float32)
    kl, z = _tc_head(aggb, hsb, eps, p)
    return (kl.reshape(N2), z.reshape(1, N2, H))


# speedup vs baseline: 1.8025x; 1.8025x over previous
import functools

import jax
import jax.numpy as jnp
from jax import lax
from jax.experimental import pallas as pl
from jax.experimental.pallas import tpu as pltpu
from jax.experimental.pallas import tpu_sc as plsc

N, E = 50000, 800000
N1, E1 = 25000, 400000
N2, E2 = 12500, 200000
H = 64

B0, CH0 = 256, 196
B1, CH1 = 256, 100
B2, CH2 = 128, 100
BG1, CHG1 = 256, 8
BG2, CHG2 = 256, 4
E0P = 16 * B0 * CH0
E1P = 16 * B1 * CH1
E2P = 16 * B2 * CH2
M1P = 16 * BG1 * CHG1
M2P = 16 * BG2 * CHG2
R0, R1, R2 = 50048, 25088, 12544
N2P = 12544

_MESH = plsc.VectorSubcoreMesh(core_axis_name="c", subcore_axis_name="s")


def _sc_segsum(table2, src, dst, zeros, *, rp, wh, b, ch, mid=None, addend=None):
    nz = rp // 16
    scratch = [
        pltpu.VMEM((b,), jnp.int32),
        pltpu.VMEM((b,), jnp.int32),
        pltpu.VMEM((b,), jnp.int32),
        pltpu.VMEM((b, wh), jnp.float32),
        pltpu.VMEM_SHARED((rp, wh), jnp.float32),
    ]
    has_add = addend is not None
    has_mid = mid is not None
    if has_add:
        scratch.append(pltpu.VMEM((b, wh), jnp.float32))
    if has_mid:
        scratch.append(pltpu.VMEM((b,), jnp.int32))
        scratch.append(pltpu.VMEM_SHARED((mid.shape[0],), jnp.int32))

    def body(*refs):
        it = iter(refs)
        tbl = next(it); srcr = next(it); dstr = next(it); zr = next(it)
        addr = next(it) if has_add else None
        midr = next(it) if has_mid else None
        out = next(it)
        sbuf = next(it); dbuf = next(it); gbuf = next(it); rows = next(it)
        acc = next(it)
        abuf = next(it) if has_add else None
        cbuf = next(it) if has_mid else None
        midsh = next(it) if has_mid else None
        c = lax.axis_index("c")
        s = lax.axis_index("s")
        pltpu.sync_copy(zr.at[pl.ds(s * nz, nz)], acc.at[pl.ds(s * nz, nz)])
        if has_mid:
            msl = mid.shape[0] // 16
            pltpu.sync_copy(midr.at[pl.ds(s * msl, msl)],
                            midsh.at[pl.ds(s * msl, msl)])
        plsc.subcore_barrier()

        def chunk(k, carry):
            base = (k * 16 + s) * b
            pltpu.sync_copy(srcr.at[pl.ds(base, b)], sbuf)
            pltpu.sync_copy(dstr.at[pl.ds(base, b)], dbuf)
            if has_mid:
                pltpu.sync_copy(midsh.at[sbuf], cbuf)
            ib_src = cbuf if has_mid else sbuf

            def ib(i, cc):
                gbuf[pl.ds(i * 16, 16)] = ib_src[pl.ds(i * 16, 16)] * 2 + c
                return cc

            lax.fori_loop(0, b // 16, ib, 0, unroll=4)
            pltpu.sync_copy(tbl.at[gbuf], rows)
            pltpu.sync_copy(rows, acc.at[dbuf], add=True)
            if has_add:
                pltpu.sync_copy(addr.at[c, pl.ds(base, b)], abuf)
                pltpu.sync_copy(abuf, acc.at[dbuf], add=True)
            return carry

        lax.fori_loop(0, ch, chunk, 0)
        plsc.subcore_barrier()
        pltpu.sync_copy(acc.at[pl.ds(s * nz, nz)], out.at[c, pl.ds(s * nz, nz)])

    args = [table2, src, dst, zeros]
    if has_add:
        args.append(addend)
    if has_mid:
        args.append(mid)
    return pl.kernel(
        body,
        out_type=jax.ShapeDtypeStruct((2, rp, wh), jnp.float32),
        mesh=_MESH,
        scratch_types=scratch,
        compiler_params=pltpu.CompilerParams(use_tc_tiling_on_sc=False, needs_layout_passes=False),
    )(*args)


def _sc_gather(table2, ids, *, wh, b, ch):
    mp = ids.shape[0]

    def body(tbl, idr, out, sbuf, gbuf, rows):
        c = lax.axis_index("c")
        s = lax.axis_index("s")

        def chunk(k, carry):
            base = (k * 16 + s) * b
            pltpu.sync_copy(idr.at[pl.ds(base, b)], sbuf)

            def ib(i, cc):
                gbuf[pl.ds(i * 16, 16)] = sbuf[pl.ds(i * 16, 16)] * 2 + c
                return cc

            lax.fori_loop(0, b // 16, ib, 0, unroll=4)
            pltpu.sync_copy(tbl.at[gbuf], rows)
            pltpu.sync_copy(rows, out.at[c, pl.ds(base, b)])
            return carry

        lax.fori_loop(0, ch, chunk, 0)

    return pl.kernel(
        body,
        out_type=jax.ShapeDtypeStruct((2, mp, wh), jnp.float32),
        mesh=_MESH,
        scratch_types=[
            pltpu.VMEM((b,), jnp.int32),
            pltpu.VMEM((b,), jnp.int32),
            pltpu.VMEM((b, wh), jnp.float32),
        ],
        compiler_params=pltpu.CompilerParams(use_tc_tiling_on_sc=False, needs_layout_passes=False),
    )(table2, ids)


def _full(a):
    return pl.BlockSpec(a.shape, lambda i: (0,) * a.ndim)


def _ln(t):
    m = jnp.mean(t, axis=-1, keepdims=True)
    v = jnp.mean((t - m) ** 2, axis=-1, keepdims=True)
    return (t - m) * lax.rsqrt(v + 1e-5)


def _mm(a, b):
    return jnp.dot(a, b, preferred_element_type=jnp.float32)


def _tc_node(x, p):
    bn = 1000
    ws = [p['Wn1'], p['bn1'].reshape(1, H), p['Wn2'], p['bn2'].reshape(1, H),
          p['Wm1_0'], p['Ws1_0'], p['Wmk_0'], p['Wsk_0']]

    def body(xr, wn1, b1, wn2, b2, wm, wsx, wmk, wsk, hr, hmr, hsr, hmkr, hskr):
        h = _mm(xr[...], wn1[...]) + b1[...]
        h = jnp.where(h > 0, h, 0.01 * h)
        h = _ln(_mm(h, wn2[...]) + b2[...])
        hr[...] = h
        hmr[...] = _mm(h, wm[...])
        hsr[...] = _mm(h, wsx[...])
        hmkr[...] = _mm(h, wmk[...])
        hskr[...] = _mm(h, wsk[...])

    return pl.pallas_call(
        body,
        grid=(N // bn,),
        in_specs=[pl.BlockSpec((bn, 16), lambda i: (i, 0))] + [_full(w) for w in ws],
        out_specs=[pl.BlockSpec((bn, H), lambda i: (i, 0)),
                   pl.BlockSpec((bn, H), lambda i: (i, 0)),
                   pl.BlockSpec((bn, H), lambda i: (i, 0)),
                   pl.BlockSpec((bn, 2 * H), lambda i: (i, 0)),
                   pl.BlockSpec((bn, 2 * H), lambda i: (i, 0))],
        out_shape=[jax.ShapeDtypeStruct((N, H), jnp.float32),
                   jax.ShapeDtypeStruct((N, H), jnp.float32),
                   jax.ShapeDtypeStruct((N, H), jnp.float32),
                   jax.ShapeDtypeStruct((N, 2 * H), jnp.float32),
                   jax.ShapeDtypeStruct((N, 2 * H), jnp.float32)],
    )(x, *ws)


def _tc_edge(edge_attr, p):
    be = 2000
    ws = [p['We1'], p['be1'].reshape(1, H), p['We2'], p['be2'].reshape(1, H),
          p['Wed_0']]

    def body(ar, w1, b1, w2, b2, wed, outr):
        e = jnp.maximum(_mm(ar[...], w1[...]) + b1[...], 0.0)
        e = _ln(_mm(e, w2[...]) + b2[...])
        em = _mm(e, wed[...])
        outr[0] = em[:, :H // 2]
        outr[1] = em[:, H // 2:]

    return pl.pallas_call(
        body,
        grid=(E // be,),
        in_specs=[pl.BlockSpec((be, 4), lambda i: (i, 0))] + [_full(w) for w in ws],
        out_specs=pl.BlockSpec((2, be, H // 2), lambda i: (0, i, 0)),
        out_shape=jax.ShapeDtypeStruct((2, E, H // 2), jnp.float32),
    )(edge_attr, *ws)


def _tc_combine(agg, hs, wlist, odims, *, n, bn, wh):
    def body(*refs):
        ar, hsr = refs[0], refs[1]
        wrs = refs[2:2 + len(wlist)]
        hr = refs[2 + len(wlist)]
        outs = refs[3 + len(wlist):]
        h = jnp.maximum(jnp.concatenate([ar[0], ar[1]], axis=-1) + hsr[...], 0.0)
        hr[...] = h
        for wr, outr in zip(wrs, outs):
            outr[...] = _mm(h, wr[...])

    w2 = 2 * wh
    return pl.pallas_call(
        body,
        grid=(n // bn,),
        in_specs=[pl.BlockSpec((2, bn, wh), lambda i: (0, i, 0)),
                  pl.BlockSpec((bn, w2), lambda i: (i, 0))] +
                 [_full(w) for w in wlist],
        out_specs=[pl.BlockSpec((bn, w2), lambda i: (i, 0))] +
                  [pl.BlockSpec((bn, od), lambda i: (i, 0)) for od in odims],
        out_shape=[jax.ShapeDtypeStruct((n, w2), jnp.float32)] +
                  [jax.ShapeDtypeStruct((n, od), jnp.float32) for od in odims],
    )(agg, hs, *wlist)


def _tc_pool(aggk, gk, agg2, g2, *, n, npad, bn, wh):
    w2 = 2 * wh

    def body(akr, gkr, a2r, g2r, tr, str_):
        i = pl.program_id(0)
        xs = jnp.maximum(jnp.concatenate([akr[0], akr[1]], -1) +
                         jnp.concatenate([gkr[0], gkr[1]], -1), 0.0)
        hb = jnp.maximum(jnp.concatenate([a2r[0], a2r[1]], -1) +
                         jnp.concatenate([g2r[0], g2r[1]], -1), 0.0)
        t = xs + hb
        tr[...] = t
        if npad != n:
            rows = lax.broadcasted_iota(jnp.int32, (bn, 1), 0) + i * bn
            t = jnp.where(rows < n, t, 0.0)

        @pl.when(i == 0)
        def _():
            str_[...] = jnp.zeros((8, w2), jnp.float32)

        str_[0:1, :] += jnp.sum(t, axis=0, keepdims=True)
        str_[1:2, :] += jnp.sum(t * t, axis=0, keepdims=True)

    eb = pl.BlockSpec((2, bn, wh), lambda i: (0, i, 0))
    return pl.pallas_call(
        body,
        grid=(npad // bn,),
        in_specs=[eb, eb, eb, eb],
        out_specs=[pl.BlockSpec((bn, w2), lambda i: (i, 0)),
                   pl.BlockSpec((8, w2), lambda i: (0, 0))],
        out_shape=[jax.ShapeDtypeStruct((npad, w2), jnp.float32),
                   jax.ShapeDtypeStruct((8, w2), jnp.float32)],
    )(aggk, gk, agg2, g2)


def _tc_bn(t, st, wlist, odims, *, n, npad, bn, w2):
    def body(*refs):
        tr, sr = refs[0], refs[1]
        wrs = refs[2:2 + len(wlist)]
        hr = refs[2 + len(wlist)]
        outs = refs[3 + len(wlist):]
        mean = sr[0:1, :] / n
        var = sr[1:2, :] / n - mean * mean
        h = jnp.maximum((tr[...] - mean) * lax.rsqrt(var + 1e-5), 0.0)
        hr[...] = h
        for wr, outr in zip(wrs, outs):
            outr[...] = _mm(h, wr[...])

    return pl.pallas_call(
        body,
        grid=(npad // bn,),
        in_specs=[pl.BlockSpec((bn, w2), lambda i: (i, 0)),
                  pl.BlockSpec((8, w2), lambda i: (0, 0))] +
                 [_full(w) for w in wlist],
        out_specs=[pl.BlockSpec((bn, w2), lambda i: (i, 0))] +
                  [pl.BlockSpec((bn, od), lambda i: (i, 0)) for od in odims],
        out_shape=[jax.ShapeDtypeStruct((npad, w2), jnp.float32)] +
                  [jax.ShapeDtypeStruct((npad, od), jnp.float32) for od in odims],
    )(t, st, *wlist)


def _tc_head(aggb, hsb, eps, p):
    bn = 784
    ws = [p['Wl'], p['bl'].reshape(1, H), p['Wmu'], p['bmu'].reshape(1, H),
          p['Wlv'], p['blv'].reshape(1, H)]

    def body(ar, hsr, er, wl, bl, wmu, bmu, wlv, blv, klr, zr):
        h5 = jnp.maximum(jnp.concatenate([ar[0], ar[1]], -1) + hsr[...], 0.0)
        xt = _mm(h5, wl[...]) + bl[...]
        mu = _ln(jnp.maximum(_mm(xt, wmu[...]) + bmu[...], 0.0))
        lv = _ln(jnp.maximum(_mm(xt, wlv[...]) + blv[...], 0.0))
        zr[...] = mu + er[...] * jnp.exp(0.5 * lv)
        kl = jnp.mean(-0.5 * (1.0 + lv - mu * mu - jnp.exp(lv)), axis=1)
        klr[...] = kl.reshape(bn, 1)

    return pl.pallas_call(
        body,
        grid=(N2P // bn,),
        in_specs=[pl.BlockSpec((2, bn, 2 * H), lambda i: (0, i, 0)),
                  pl.BlockSpec((bn, 4 * H), lambda i: (i, 0)),
                  pl.BlockSpec((bn, H), lambda i: (i, 0))] +
                 [_full(w) for w in ws],
        out_specs=[pl.BlockSpec((bn, 1), lambda i: (i, 0)),
                   pl.BlockSpec((bn, H), lambda i: (i, 0))],
        out_shape=[jax.ShapeDtypeStruct((N2P, 1), jnp.float32),
                   jax.ShapeDtypeStruct((N2P, H), jnp.float32)],
    )(aggb, hsb, eps, *ws)


def _pad1(a, length, val):
    return jnp.concatenate(
        [a, jnp.full((length - a.shape[0],), val, a.dtype)])


def kernel(x, edge_attr, weights, params, edge_index, m_id0, m_g1, m_id1, m_g2):
    p = params
    src0 = _pad1(edge_index[0], E0P, 0)
    dst0 = _pad1(edge_index[1], E0P, N)
    src1 = _pad1(m_g1[0], E1P, 0)
    dst1 = _pad1(m_g1[1], E1P, N1)
    src2 = _pad1(m_g2[0], E2P, 0)
    dst2 = _pad1(m_g2[1], E2P, N2)
    mid0p = _pad1(m_id0, M1P, 0)
    mid1p = _pad1(m_id1, M2P, 0)
    z0 = jnp.zeros((R0, H // 2), jnp.float32)
    z1 = jnp.zeros((R1, H), jnp.float32)
    z2 = jnp.zeros((R2, 2 * H), jnp.float32)

    h, hm0, hs0, hmk, hsk = _tc_node(x, p)
    emsg = _tc_edge(edge_attr, p)

    agg0 = _sc_segsum(hm0.reshape(2 * N, H // 2), src0, dst0, z0,
                      rp=R0, wh=H // 2, b=B0, ch=CH0, addend=emsg)
    h1, hm2, hs2 = _tc_combine(
        agg0, hs0, [p['Wm2_0'], p['Ws2_0']], [2 * H, 2 * H],
        n=N, bn=1000, wh=H // 2)

    aggk = _sc_segsum(hmk.reshape(2 * N, H), src1, dst1, z1,
                      rp=R1, wh=H, b=B1, ch=CH1, mid=mid0p)
    agg2 = _sc_segsum(hm2.reshape(2 * N, H), src1, dst1, z1,
                      rp=R1, wh=H, b=B1, ch=CH1, mid=mid0p)
    gk = _sc_gather(hsk.reshape(2 * N, H), mid0p, wh=H, b=BG1, ch=CHG1)
    g2 = _sc_gather(hs2.reshape(2 * N, H), mid0p, wh=H, b=BG1, ch=CHG1)
    t, st = _tc_pool(aggk, gk, agg2, g2, n=N1, npad=N1, bn=1000, wh=H)
    h2, hm11, hs11, hmk1, hsk1 = _tc_bn(
        t, st, [p['Wm1_1'], p['Ws1_1'], p['Wmk_1'], p['Wsk_1']],
        [2 * H, 2 * H, 4 * H, 4 * H], n=N1, npad=N1, bn=1000, w2=2 * H)

    agg11 = _sc_segsum(hm11.reshape(2 * N1, H), src1, dst1, z1,
                       rp=R1, wh=H, b=B1, ch=CH1)
    h3, hm21, hs21 = _tc_combine(
        agg11, hs11, [p['Wm2_1'], p['Ws2_1']], [4 * H, 4 * H],
        n=N1, bn=1000, wh=H)

    aggk1 = _sc_segsum(hmk1.reshape(2 * N1, 2 * H), src2, dst2, z2,
                       rp=R2, wh=2 * H, b=B2, ch=CH2, mid=mid1p)
    agg21 = _sc_segsum(hm21.reshape(2 * N1, 2 * H), src2, dst2, z2,
                       rp=R2, wh=2 * H, b=B2, ch=CH2, mid=mid1p)
    gk1 = _sc_gather(hsk1.reshape(2 * N1, 2 * H), mid1p, wh=2 * H, b=BG2, ch=CHG2)
    g21 = _sc_gather(hs21.reshape(2 * N1, 2 * H), mid1p, wh=2 * H, b=BG2, ch=CHG2)
    t2, st2 = _tc_pool(aggk1, gk1, agg21, g21, n=N2, npad=N2P, bn=784, wh=2 * H)
    h4, hmb, hsb = _tc_bn(
        t2, st2, [p['Wmb'], p['Wsb']], [4 * H, 4 * H],
        n=N2, npad=N2P, bn=784, w2=4 * H)

    aggb = _sc_segsum(hmb.reshape(2 * N2P, 2 * H), src2, dst2, z2,
                      rp=R2, wh=2 * H, b=B2, ch=CH2)

    eps = jnp.concatenate(
        [jax.random.normal(jax.random.key(42), (N2, H), jnp.float32),
         jnp.zeros((N2P - N2, H), jnp.float32)])
    kl, z = _tc_head(aggb, hsb, eps, p)
    return (kl[:N2].reshape(N2), z[:N2].reshape(1, N2, H))
